# Initial kernel scaffold; baseline (speedup 1.0000x reference)
import jax
import jax.numpy as jnp
from jax.experimental import pallas as pl


def kernel(proposals, objectness):
    raise NotImplementedError("write your pallas kernel here")


# speedup vs baseline: 14.4057x; 14.4057x over previous
import jax
import jax.numpy as jnp
from jax.experimental import pallas as pl
from jax.experimental.pallas import tpu as pltpu

IMG = 800.0
PRE = 2000
POST = 1000
THR = 0.7
MINSZ = 1e-3
NPAD = 2048
T = 256
NT = NPAD // T


def _nms_body(logit_ref, rows_ref, cols_ref, score_out, boxes_out, m_ref, keep_ref):
    bc_clip = jnp.clip(cols_ref[...], 0.0, IMG)
    boxes_out[...] = bc_clip

    br = jnp.clip(rows_ref[...], 0.0, IMG)
    x1r = br[0:1, :]
    y1r = br[1:2, :]
    x2r = br[2:3, :]
    y2r = br[3:4, :]
    wr = x2r - x1r
    hr = y2r - y1r
    area_r = wr * hr
    scores = jax.nn.sigmoid(logit_ref[...])
    valid = (wr >= MINSZ) & (hr >= MINSZ) & (scores >= 0.0)

    keep_ref[...] = jnp.ones((1, NPAD), jnp.float32)
    giota = jax.lax.broadcasted_iota(jnp.int32, (1, NPAD), 1)
    li = jax.lax.broadcasted_iota(jnp.int32, (1, T), 1)
    tri = (jax.lax.broadcasted_iota(jnp.int32, (T, T), 1)
           > jax.lax.broadcasted_iota(jnp.int32, (T, T), 0))

    for t in range(NT):
        base = t * T
        xc1 = bc_clip[base:base + T, 0:1]
        yc1 = bc_clip[base:base + T, 1:2]
        xc2 = bc_clip[base:base + T, 2:3]
        yc2 = bc_clip[base:base + T, 3:4]
        area_c = (xc2 - xc1) * (yc2 - yc1)

        wij = jnp.clip(jnp.minimum(xc2, x2r[:, base:base + T])
                       - jnp.maximum(xc1, x1r[:, base:base + T]), 0.0, None)
        hij = jnp.clip(jnp.minimum(yc2, y2r[:, base:base + T])
                       - jnp.maximum(yc1, y1r[:, base:base + T]), 0.0, None)
        inter = wij * hij
        iou = inter / (area_c + area_r[:, base:base + T] - inter + 1e-9)
        m_ref[...] = jnp.where((iou > THR) & tri, 1.0, 0.0)

        def body(i, alive):
            row = m_ref[pl.ds(i, 1), :]
            a_i = jnp.sum(jnp.where(li == i, alive, 0.0))
            return alive * (1.0 - row * a_i)

        alive = jax.lax.fori_loop(0, T, body, keep_ref[0:1, base:base + T])
        keep_ref[0:1, base:base + T] = alive

        if t < NT - 1:
            wb = jnp.clip(jnp.minimum(xc2, x2r) - jnp.maximum(xc1, x1r), 0.0, None)
            hb = jnp.clip(jnp.minimum(yc2, y2r) - jnp.maximum(yc1, y1r), 0.0, None)
            interb = wb * hb
            ioub = interb / (area_c + area_r - interb + 1e-9)
            mb = jnp.where(ioub > THR, 1.0, 0.0)
            cnt = jnp.dot(alive, mb, preferred_element_type=jnp.float32)
            supp = (cnt > 0.5) & (giota >= base + T)
            keep_ref[...] = jnp.where(supp, 0.0, keep_ref[...])

    final_mask = (keep_ref[...] > 0.5) & valid
    score_out[...] = jnp.where(final_mask, scores, -1.0)


def kernel(proposals, objectness):
    obj = objectness.reshape(objectness.shape[0], -1)
    scores0 = obj[0]
    boxes0 = proposals[0]
    top_vals, top_idx = jax.lax.top_k(scores0, PRE)
    boxes = jnp.take(boxes0, top_idx, axis=0)
    cols = jnp.pad(boxes, ((0, NPAD - PRE), (0, 0)))
    rows = cols.T
    logits = jnp.pad(top_vals, (0, NPAD - PRE), constant_values=-1e4)[None, :]

    nms_scores, boxes_clip = pl.pallas_call(
        _nms_body,
        out_shape=[jax.ShapeDtypeStruct((1, NPAD), jnp.float32),
                   jax.ShapeDtypeStruct((NPAD, 4), jnp.float32)],
        scratch_shapes=[pltpu.VMEM((T, T), jnp.float32),
                        pltpu.VMEM((1, NPAD), jnp.float32)],
    )(logits, rows, cols)

    final_scores, final_idx = jax.lax.top_k(nms_scores[0, :PRE], POST)
    final_boxes = boxes_clip[:PRE][final_idx]
    return final_boxes, final_scores


# speedup vs baseline: 60.4667x; 4.1974x over previous
import jax
import jax.numpy as jnp
from jax.experimental import pallas as pl
from jax.experimental.pallas import tpu as pltpu

IMG = 800.0
PRE = 2000
POST = 1000
THR = 0.7
MINSZ = 1e-3
NPAD = 2048
T = 256
NT = NPAD // T


def _nms_body(logit_ref, rows_ref, cols_ref, score_out, boxes_out, keep_ref):
    bc_clip = jnp.clip(cols_ref[...], 0.0, IMG)
    boxes_out[...] = bc_clip

    br = jnp.clip(rows_ref[...], 0.0, IMG)
    x1r = br[0:1, :]
    y1r = br[1:2, :]
    x2r = br[2:3, :]
    y2r = br[3:4, :]
    area_r = (x2r - x1r) * (y2r - y1r)

    xc1 = bc_clip[:, 0:1]
    yc1 = bc_clip[:, 1:2]
    xc2 = bc_clip[:, 2:3]
    yc2 = bc_clip[:, 3:4]
    wc = xc2 - xc1
    hc = yc2 - yc1
    area_c = wc * hc

    scores = jax.nn.sigmoid(logit_ref[...])
    valid = (wc >= MINSZ) & (hc >= MINSZ) & (scores >= 0.0)

    keep_ref[...] = jnp.ones((NPAD, 1), jnp.float32)
    gi_col = jax.lax.broadcasted_iota(jnp.int32, (NPAD, 1), 0)

    for t in range(NT):
        base = t * T
        x1t = x1r[:, base:base + T]
        y1t = y1r[:, base:base + T]
        x2t = x2r[:, base:base + T]
        y2t = y2r[:, base:base + T]
        at = area_r[:, base:base + T]
        wij = jnp.clip(jnp.minimum(xc2, x2t) - jnp.maximum(xc1, x1t), 0.0, None)
        hij = jnp.clip(jnp.minimum(yc2, y2t) - jnp.maximum(yc1, y1t), 0.0, None)
        inter = wij * hij
        iou = inter / (area_c + at - inter + 1e-9)
        li = jax.lax.broadcasted_iota(jnp.int32, (NPAD, T), 1) + base
        C = jnp.where((iou > THR) & (li < gi_col), 1.0, 0.0)

        B = C[base:base + T, :]
        a0 = keep_ref[base:base + T, :]

        def step(a):
            cnt = jnp.dot(B, a, preferred_element_type=jnp.float32)
            return jnp.where(cnt > 0.5, 0.0, a0)

        def cond(state):
            prev, cur = state
            return jnp.any(prev != cur)

        def body(state):
            _, cur = state
            return cur, step(cur)

        _, a_t = jax.lax.while_loop(cond, body, (a0, step(a0)))
        keep_ref[base:base + T, :] = a_t

        if t < NT - 1:
            cnt = jnp.dot(C, a_t, preferred_element_type=jnp.float32)
            supp = (cnt > 0.5) & (gi_col >= base + T)
            keep_ref[...] = jnp.where(supp, 0.0, keep_ref[...])

    final_mask = (keep_ref[...] > 0.5) & valid
    score_out[...] = jnp.where(final_mask, scores, -1.0)


def kernel(proposals, objectness):
    obj = objectness.reshape(objectness.shape[0], -1)
    scores0 = obj[0]
    boxes0 = proposals[0]
    top_vals, top_idx = jax.lax.top_k(scores0, PRE)
    boxes = jnp.take(boxes0, top_idx, axis=0)
    cols = jnp.pad(boxes, ((0, NPAD - PRE), (0, 0)))
    rows = cols.T
    logits = jnp.pad(top_vals, (0, NPAD - PRE), constant_values=-1e4)[:, None]

    nms_scores, boxes_clip = pl.pallas_call(
        _nms_body,
        out_shape=[jax.ShapeDtypeStruct((NPAD, 1), jnp.float32),
                   jax.ShapeDtypeStruct((NPAD, 4), jnp.float32)],
        scratch_shapes=[pltpu.VMEM((NPAD, 1), jnp.float32)],
    )(logits, rows, cols)

    final_scores, final_idx = jax.lax.top_k(nms_scores[:PRE, 0], POST)
    final_boxes = boxes_clip[:PRE][final_idx]
    return final_boxes, final_scores
